# baseline (speedup 1.0000x reference)
import jax
import jax.numpy as jnp
from jax.experimental import pallas as pl
from jax.experimental.pallas import tpu as pltpu

NUM_CH = 192
ROWS = 392
LANES = 128


def _copy_body(perm_ref, x_ref, o_ref):
    o_ref[...] = x_ref[...]


def kernel(x, permutation):
    b, c, h, w = x.shape
    xr = x.reshape(b, c, ROWS, LANES)
    grid_spec = pltpu.PrefetchScalarGridSpec(
        num_scalar_prefetch=1,
        grid=(b, c),
        in_specs=[
            pl.BlockSpec((1, 1, ROWS, LANES), lambda i, j, perm: (i, perm[j], 0, 0)),
        ],
        out_specs=pl.BlockSpec((1, 1, ROWS, LANES), lambda i, j, perm: (i, j, 0, 0)),
    )
    out = pl.pallas_call(
        _copy_body,
        grid_spec=grid_spec,
        out_shape=jax.ShapeDtypeStruct((b, c, ROWS, LANES), x.dtype),
    )(permutation.astype(jnp.int32), xr)
    return out.reshape(b, c, h, w)
